# baseline (speedup 1.0000x reference)
import dataclasses

import jax
import jax.numpy as jnp
from jax import lax
from jax.experimental import pallas as pl
from jax.experimental.pallas import tpu as pltpu
from jax.experimental.pallas import tpu_sc as plsc

_NC, _NS, _L = 2, 16, 16
_NW = _NC * _NS
_BATCH = 16384
_D = 64
_BW = _BATCH // _NW
_CH = 128
_NCH = _BW // _CH


def _vsqrt(x):
    i = plsc.bitcast(x, jnp.int32)
    y = plsc.bitcast(jnp.int32(0x5F3759DF) - (i >> 1), jnp.float32)
    for _ in range(3):
        y = y * (1.5 - 0.5 * x * y * y)
    return x * y


def _body(heads_hbm, rels_hbm, tails_hbm, ent_hbm, rel_hbm, out_hbm,
          hidx_v, ridx_v, tidx_v, h_v, r_v, t_v, sq_v, s_v, sem):
    wid = lax.axis_index("s") * _NC + lax.axis_index("c")
    base = wid * _BW

    @pl.loop(0, _NCH)
    def _chunk(c):
        off = base + c * _CH
        pltpu.sync_copy(heads_hbm.at[pl.ds(off, _CH)], hidx_v)
        pltpu.sync_copy(rels_hbm.at[pl.ds(off, _CH)], ridx_v)
        pltpu.sync_copy(tails_hbm.at[pl.ds(off, _CH)], tidx_v)
        ch = pltpu.async_copy(ent_hbm.at[hidx_v], h_v, sem)
        cr = pltpu.async_copy(rel_hbm.at[ridx_v], r_v, sem)
        ct = pltpu.async_copy(ent_hbm.at[tidx_v], t_v, sem)
        ch.wait()
        cr.wait()
        ct.wait()

        @pl.loop(0, _CH)
        def _row(i):
            acc = jnp.zeros((_L,), jnp.float32)
            for j in range(_D // _L):
                sl = pl.ds(j * _L, _L)
                d = h_v[i, sl] + r_v[i, sl] - t_v[i, sl]
                acc = acc + d * d
            sq_v[i, :] = acc

        lanes = lax.iota(jnp.int32, _L)

        @pl.loop(0, _CH, step=_L)
        def _grp(i0):
            rows = i0 + lanes
            tot = jnp.zeros((_L,), jnp.float32)
            for col in range(_L):
                cols = jnp.full((_L,), col, jnp.int32)
                tot = tot + plsc.load_gather(sq_v, [rows, cols])
            s_v[pl.ds(i0, _L)] = _vsqrt(tot)

        pltpu.sync_copy(s_v, out_hbm.at[pl.ds(off, _CH)])


@jax.jit
def kernel(heads, relations, tails, entity_emb, relation_emb):
    mesh = plsc.VectorSubcoreMesh(core_axis_name="c", subcore_axis_name="s")
    cp = pltpu.CompilerParams()
    if "needs_layout_passes" in pltpu.CompilerParams.__dataclass_fields__:
        cp = dataclasses.replace(cp, needs_layout_passes=False)
    if "use_tc_tiling_on_sc" in pltpu.CompilerParams.__dataclass_fields__:
        cp = dataclasses.replace(cp, use_tc_tiling_on_sc=False)
    run = pl.kernel(
        _body,
        out_type=jax.ShapeDtypeStruct((_BATCH,), jnp.float32),
        mesh=mesh,
        scratch_types=[
            pltpu.VMEM((_CH,), jnp.int32),
            pltpu.VMEM((_CH,), jnp.int32),
            pltpu.VMEM((_CH,), jnp.int32),
            pltpu.VMEM((_CH, _D), jnp.float32),
            pltpu.VMEM((_CH, _D), jnp.float32),
            pltpu.VMEM((_CH, _D), jnp.float32),
            pltpu.VMEM((_CH, _L), jnp.float32),
            pltpu.VMEM((_CH,), jnp.float32),
            pltpu.SemaphoreType.DMA,
        ],
        compiler_params=cp,
    )
    return run(heads, relations, tails, entity_emb, relation_emb)
